# baseline (speedup 1.0000x reference)
import jax
import jax.numpy as jnp
from jax import lax
from jax.experimental import pallas as pl
from jax.experimental.pallas import tpu as pltpu
from jax.experimental.pallas import tpu_sc as plsc

N = 10000
E = 320000
G = 10
NPG = 1000
F0, F1, F2 = 128, 32, 64
FW = 128
FA = 128

NCORES = 2
NSUB = 16
NTILES = NCORES * NSUB
CHUNK = 128
NBUF = 2
SLAB = 16
NSLAB = 5
NCHUNK = 80
EPT = NCHUNK * CHUNK
ETOT = NTILES * EPT
NPAD = 10240
RPT = NPAD // NSUB


def _sc_mesh():
    return plsc.VectorSubcoreMesh(core_axis_name="c", subcore_axis_name="s")


def _deg_body(dst_hbm, zeros_hbm, out_hbm, idx_v, ones_v, acc):
    cid = lax.axis_index("c")
    sid = lax.axis_index("s")
    wid = sid * NCORES + cid
    pltpu.sync_copy(zeros_hbm.at[pl.ds(sid * RPT, RPT)],
                    acc.at[pl.ds(sid * RPT, RPT)])
    for i in range(CHUNK // 16):
        ones_v[pl.ds(i * 16, 16)] = jnp.ones((16,), jnp.float32)
    pltpu.sync_copy(dst_hbm.at[wid], idx_v)
    plsc.subcore_barrier()

    @pl.loop(0, NCHUNK)
    def _chunk(j):
        pltpu.sync_copy(ones_v, acc.at[idx_v.at[j]], add=True)

    plsc.subcore_barrier()
    pltpu.sync_copy(acc.at[pl.ds(sid * RPT, RPT)],
                    out_hbm.at[cid, pl.ds(sid * RPT, RPT)])


def _make_deg_kernel():
    return pl.kernel(
        _deg_body,
        out_type=jax.ShapeDtypeStruct((NCORES, NPAD), jnp.float32),
        mesh=_sc_mesh(),
        scratch_types=[
            pltpu.VMEM((NCHUNK, CHUNK), jnp.int32),
            pltpu.VMEM((CHUNK,), jnp.float32),
            pltpu.VMEM_SHARED((NPAD,), jnp.float32),
        ],
    )


def _scatter_body(src_hbm, dst_hbm, ht_hbm, zeros_hbm, out_hbm,
                  sv0, sv1, dv0, dv1, b0, b1,
                  g0, g1, s0, s1, isv0, isv1, idv0, idv1, acc):
    svs = (sv0, sv1)
    dvs = (dv0, dv1)
    bufs = (b0, b1)
    gsems = (g0, g1)
    ssems = (s0, s1)
    isvs = (isv0, isv1)
    idvs = (idv0, idv1)
    cid = lax.axis_index("c")
    sid = lax.axis_index("s")
    wid = sid * NCORES + cid

    def idx_load(s):
        par = s % 2
        sl = pl.ds(s * SLAB, SLAB)
        return (pltpu.make_async_copy(src_hbm.at[wid, sl], svs[par], isvs[par]),
                pltpu.make_async_copy(dst_hbm.at[wid, sl], dvs[par], idvs[par]))

    def gather(j):
        s, k = divmod(j, SLAB)
        return pltpu.make_async_copy(
            ht_hbm.at[svs[s % 2].at[k]], bufs[j % NBUF], gsems[j % NBUF])

    def scatter(j):
        s, k = divmod(j, SLAB)
        return pltpu.make_async_copy(
            bufs[j % NBUF], acc.at[dvs[s % 2].at[k]], ssems[j % NBUF])

    pltpu.sync_copy(zeros_hbm.at[pl.ds(sid * RPT, RPT)],
                    acc.at[pl.ds(sid * RPT, RPT)])
    for cp in idx_load(0):
        cp.start()
    for cp in idx_load(0):
        cp.wait()
    plsc.subcore_barrier()

    gather(0).start()
    for cp in idx_load(1):
        cp.start()

    for j in range(NCHUNK):
        s, k = divmod(j, SLAB)
        if k == SLAB - 1 and s + 1 < NSLAB:
            for cp in idx_load(s + 1):
                cp.wait()
        if k == NBUF and 1 <= s and s + 1 < NSLAB:
            for cp in idx_load(s + 1):
                cp.start()
        gather(j).wait()
        scatter(j).start(add=True)
        if j >= 1:
            scatter(j - 1).wait()
        if j + 1 < NCHUNK:
            gather(j + 1).start()
    scatter(NCHUNK - 1).wait()

    plsc.subcore_barrier()
    pltpu.sync_copy(acc.at[pl.ds(sid * RPT, RPT)],
                    out_hbm.at[cid, pl.ds(sid * RPT, RPT)])


def _make_scatter_kernel():
    return pl.kernel(
        _scatter_body,
        out_type=jax.ShapeDtypeStruct((NCORES, NPAD, FA), jnp.float32),
        mesh=_sc_mesh(),
        scratch_types=[
            pltpu.VMEM((SLAB, CHUNK), jnp.int32),
            pltpu.VMEM((SLAB, CHUNK), jnp.int32),
            pltpu.VMEM((SLAB, CHUNK), jnp.int32),
            pltpu.VMEM((SLAB, CHUNK), jnp.int32),
            pltpu.VMEM((CHUNK, FW), jnp.float32),
            pltpu.VMEM((CHUNK, FW), jnp.float32),
            pltpu.SemaphoreType.DMA,
            pltpu.SemaphoreType.DMA,
            pltpu.SemaphoreType.DMA,
            pltpu.SemaphoreType.DMA,
            pltpu.SemaphoreType.DMA,
            pltpu.SemaphoreType.DMA,
            pltpu.SemaphoreType.DMA,
            pltpu.SemaphoreType.DMA,
            pltpu.VMEM_SHARED((NPAD, FA), jnp.float32),
        ],
    )


def _k2_body(x_ref, dp_ref, w1_ref, ht_ref, dinv_ref):
    deg = dp_ref[0] + dp_ref[1] + 1.0
    dinv = 1.0 / jnp.sqrt(deg)
    h1 = jnp.dot(x_ref[...].astype(jnp.bfloat16),
                 w1_ref[...].astype(jnp.bfloat16),
                 preferred_element_type=jnp.float32)
    ht_ref[...] = jnp.concatenate(
        [h1 * dinv, jnp.zeros((NPG, FW - F1), jnp.float32)], axis=1)
    dinv_ref[...] = dinv


def _k4_body(p_ref, ht_ref, dinv_ref, b1_ref, w2_ref, out_ref):
    dinv = dinv_ref[...]
    agg = (p_ref[0] + p_ref[1])[:, :F1] + ht_ref[:, :F1]
    z1 = jnp.maximum(dinv * agg + b1_ref[...], 0.0)
    ht2 = jnp.dot(z1.astype(jnp.bfloat16), w2_ref[...].astype(jnp.bfloat16),
                  preferred_element_type=jnp.float32) * dinv
    out_ref[...] = jnp.concatenate(
        [ht2, jnp.zeros((NPG, FW - F2), jnp.float32)], axis=1)


def _k6_body(p_ref, ht_ref, dinv_ref, bc2_ref, w1t_ref, bf1_ref, wf2_ref,
             bf2_ref, out_ref):
    agg = (p_ref[0] + p_ref[1])[:, :F2] + ht_ref[:, :F2]
    h = dinv_ref[...] * agg + bc2_ref[...]
    h = jnp.maximum(h, 0.0)
    ctx = jnp.sum(h, axis=0, keepdims=True) * (1.0 / NPG)
    se = jnp.concatenate(
        [h[0:1], h[250:251], h[500:501], h[999:1000]], axis=0)
    comb = jnp.concatenate([se, jnp.broadcast_to(ctx, (4, F2))], axis=1)
    a = jnp.dot(comb.astype(jnp.bfloat16), w1t_ref[...].astype(jnp.bfloat16),
                preferred_element_type=jnp.float32)
    a = jnp.maximum(a + bf1_ref[...], 0.0)
    ab = a.astype(jnp.bfloat16).astype(jnp.float32)
    wb = wf2_ref[...].astype(jnp.bfloat16).astype(jnp.float32)
    q = jnp.sum(ab * wb, axis=1)
    g = pl.program_id(0)
    out_ref[pl.ds(g, 1), :] = q[None, :] + bf2_ref[...]


def kernel(x, edge_index, batch, Wc1, bc1, Wc2, bc2, Wf1, bf1, Wf2, bf2):
    del batch
    src = edge_index[0]
    dst = edge_index[1]
    pad = ETOT - E
    ar = jnp.arange(pad, dtype=jnp.int32)
    srcp = jnp.concatenate([src, ar % 128]).reshape(NTILES, NCHUNK, CHUNK)
    dstp = jnp.concatenate([dst, N + (ar % 128)]).reshape(NTILES, NCHUNK, CHUNK)
    zeros1 = jnp.zeros((NPAD,), jnp.float32)
    zerosa = jnp.zeros((NPAD, FA), jnp.float32)

    degp = _make_deg_kernel()(dstp, zeros1)

    ht1, dinv = pl.pallas_call(
        _k2_body,
        grid=(G,),
        in_specs=[
            pl.BlockSpec((NPG, F0), lambda g: (g, 0)),
            pl.BlockSpec((NCORES, NPG, 1), lambda g: (0, g, 0)),
            pl.BlockSpec((F0, F1), lambda g: (0, 0)),
        ],
        out_specs=[
            pl.BlockSpec((NPG, FW), lambda g: (g, 0)),
            pl.BlockSpec((NPG, 1), lambda g: (g, 0)),
        ],
        out_shape=[
            jax.ShapeDtypeStruct((N, FW), jnp.float32),
            jax.ShapeDtypeStruct((N, 1), jnp.float32),
        ],
    )(x, degp.reshape(NCORES, NPAD, 1), Wc1)

    p1 = _make_scatter_kernel()(srcp, dstp, ht1, zerosa)

    ht2 = pl.pallas_call(
        _k4_body,
        grid=(G,),
        in_specs=[
            pl.BlockSpec((NCORES, NPG, FA), lambda g: (0, g, 0)),
            pl.BlockSpec((NPG, FW), lambda g: (g, 0)),
            pl.BlockSpec((NPG, 1), lambda g: (g, 0)),
            pl.BlockSpec((1, F1), lambda g: (0, 0)),
            pl.BlockSpec((F1, F2), lambda g: (0, 0)),
        ],
        out_specs=pl.BlockSpec((NPG, FW), lambda g: (g, 0)),
        out_shape=jax.ShapeDtypeStruct((N, FW), jnp.float32),
    )(p1, ht1, dinv, bc1.reshape(1, F1), Wc2)

    p2 = _make_scatter_kernel()(srcp, dstp, ht2, zerosa)

    out = pl.pallas_call(
        _k6_body,
        grid=(G,),
        in_specs=[
            pl.BlockSpec((NCORES, NPG, FA), lambda g: (0, g, 0)),
            pl.BlockSpec((NPG, FW), lambda g: (g, 0)),
            pl.BlockSpec((NPG, 1), lambda g: (g, 0)),
            pl.BlockSpec((1, F2), lambda g: (0, 0)),
            pl.BlockSpec((F0, F2), lambda g: (0, 0)),
            pl.BlockSpec((1, F2), lambda g: (0, 0)),
            pl.BlockSpec((1, F2), lambda g: (0, 0)),
            pl.BlockSpec((1, 1), lambda g: (0, 0)),
        ],
        out_specs=pl.BlockSpec((G, 4), lambda g: (0, 0)),
        out_shape=jax.ShapeDtypeStruct((G, 4), jnp.float32),
    )(p2, ht2, dinv, bc2.reshape(1, F2), Wf1.T, bf1.reshape(1, F2),
      Wf2, bf2.reshape(1, 1))

    return out


# speedup vs baseline: 1.1400x; 1.1400x over previous
import jax
import jax.numpy as jnp
from jax import lax
from jax.experimental import pallas as pl
from jax.experimental.pallas import tpu as pltpu
from jax.experimental.pallas import tpu_sc as plsc

N = 10000
E = 320000
G = 10
NPG = 1000
F0, F1, F2 = 128, 32, 64
FW = 128
FA = 128

NCORES = 2
NSUB = 16
NTILES = NCORES * NSUB
CHUNK = 128
NBUF = 2
SLAB = 16
NSLAB = 5
NCHUNK = 80
EPT = NCHUNK * CHUNK
ETOT = NTILES * EPT
NPAD = 10240
RPT = NPAD // NSUB


def _sc_mesh():
    return plsc.VectorSubcoreMesh(core_axis_name="c", subcore_axis_name="s")


def _deg_body(dst_hbm, zeros_hbm, out_hbm, idx_v, ones_v, acc):
    cid = lax.axis_index("c")
    sid = lax.axis_index("s")
    wid = sid * NCORES + cid
    pltpu.sync_copy(zeros_hbm.at[pl.ds(sid * RPT, RPT)],
                    acc.at[pl.ds(sid * RPT, RPT)])
    for i in range(CHUNK // 16):
        ones_v[pl.ds(i * 16, 16)] = jnp.ones((16,), jnp.float32)
    pltpu.sync_copy(dst_hbm.at[wid], idx_v)
    plsc.subcore_barrier()

    @pl.loop(0, NCHUNK)
    def _chunk(j):
        pltpu.sync_copy(ones_v, acc.at[idx_v.at[j]], add=True)

    plsc.subcore_barrier()
    pltpu.sync_copy(acc.at[pl.ds(sid * RPT, RPT)],
                    out_hbm.at[cid, pl.ds(sid * RPT, RPT)])


def _make_deg_kernel():
    return pl.kernel(
        _deg_body,
        out_type=jax.ShapeDtypeStruct((NCORES, NPAD), jnp.float32),
        mesh=_sc_mesh(),
        scratch_types=[
            pltpu.VMEM((NCHUNK, CHUNK), jnp.int32),
            pltpu.VMEM((CHUNK,), jnp.float32),
            pltpu.VMEM_SHARED((NPAD,), jnp.float32),
        ],
    )


def _scatter_body(src_hbm, dst_hbm, ht_hbm, zeros_hbm, out_hbm,
                  sv0, sv1, dv0, dv1, b0, b1,
                  g0, g1, s0, s1, isv0, isv1, idv0, idv1, acc):
    svs = (sv0, sv1)
    dvs = (dv0, dv1)
    bufs = (b0, b1)
    gsems = (g0, g1)
    ssems = (s0, s1)
    isvs = (isv0, isv1)
    idvs = (idv0, idv1)
    cid = lax.axis_index("c")
    sid = lax.axis_index("s")
    wid = sid * NCORES + cid

    def idx_load(s):
        par = s % 2
        sl = pl.ds(s * SLAB, SLAB)
        return (pltpu.make_async_copy(src_hbm.at[wid, sl], svs[par], isvs[par]),
                pltpu.make_async_copy(dst_hbm.at[wid, sl], dvs[par], idvs[par]))

    def gather(j):
        s, k = divmod(j, SLAB)
        return pltpu.make_async_copy(
            ht_hbm.at[svs[s % 2].at[k]], bufs[j % NBUF], gsems[j % NBUF])

    def scatter(j):
        s, k = divmod(j, SLAB)
        return pltpu.make_async_copy(
            bufs[j % NBUF], acc.at[dvs[s % 2].at[k]], ssems[j % NBUF])

    pltpu.sync_copy(zeros_hbm.at[pl.ds(sid * RPT, RPT)],
                    acc.at[pl.ds(sid * RPT, RPT)])
    for cp in idx_load(0):
        cp.start()
    for cp in idx_load(0):
        cp.wait()
    plsc.subcore_barrier()

    gather(0).start()
    for cp in idx_load(1):
        cp.start()

    for j in range(NCHUNK):
        s, k = divmod(j, SLAB)
        if k == SLAB - 1 and s + 1 < NSLAB:
            for cp in idx_load(s + 1):
                cp.wait()
        if k == NBUF and 1 <= s and s + 1 < NSLAB:
            for cp in idx_load(s + 1):
                cp.start()
        if j >= 1:
            scatter(j - 1).wait()
        if j + 1 < NCHUNK:
            gather(j + 1).start()
        gather(j).wait()
        scatter(j).start(add=True)
    scatter(NCHUNK - 1).wait()

    plsc.subcore_barrier()
    pltpu.sync_copy(acc.at[pl.ds(sid * RPT, RPT)],
                    out_hbm.at[cid, pl.ds(sid * RPT, RPT)])


def _make_scatter_kernel():
    return pl.kernel(
        _scatter_body,
        out_type=jax.ShapeDtypeStruct((NCORES, NPAD, FA), jnp.float32),
        mesh=_sc_mesh(),
        scratch_types=[
            pltpu.VMEM((SLAB, CHUNK), jnp.int32),
            pltpu.VMEM((SLAB, CHUNK), jnp.int32),
            pltpu.VMEM((SLAB, CHUNK), jnp.int32),
            pltpu.VMEM((SLAB, CHUNK), jnp.int32),
            pltpu.VMEM((CHUNK, FW), jnp.float32),
            pltpu.VMEM((CHUNK, FW), jnp.float32),
            pltpu.SemaphoreType.DMA,
            pltpu.SemaphoreType.DMA,
            pltpu.SemaphoreType.DMA,
            pltpu.SemaphoreType.DMA,
            pltpu.SemaphoreType.DMA,
            pltpu.SemaphoreType.DMA,
            pltpu.SemaphoreType.DMA,
            pltpu.SemaphoreType.DMA,
            pltpu.VMEM_SHARED((NPAD, FA), jnp.float32),
        ],
    )


def _k2_body(x_ref, dp_ref, w1_ref, ht_ref, dinv_ref):
    deg = dp_ref[0] + dp_ref[1] + 1.0
    dinv = 1.0 / jnp.sqrt(deg)
    h1 = jnp.dot(x_ref[...].astype(jnp.bfloat16),
                 w1_ref[...].astype(jnp.bfloat16),
                 preferred_element_type=jnp.float32)
    ht_ref[...] = jnp.concatenate(
        [h1 * dinv, jnp.zeros((NPG, FW - F1), jnp.float32)], axis=1)
    dinv_ref[...] = dinv


def _k4_body(p_ref, ht_ref, dinv_ref, b1_ref, w2_ref, out_ref):
    dinv = dinv_ref[...]
    agg = (p_ref[0] + p_ref[1])[:, :F1] + ht_ref[:, :F1]
    z1 = jnp.maximum(dinv * agg + b1_ref[...], 0.0)
    ht2 = jnp.dot(z1.astype(jnp.bfloat16), w2_ref[...].astype(jnp.bfloat16),
                  preferred_element_type=jnp.float32) * dinv
    out_ref[...] = jnp.concatenate(
        [ht2, jnp.zeros((NPG, FW - F2), jnp.float32)], axis=1)


def _k6_body(p_ref, ht_ref, dinv_ref, bc2_ref, w1t_ref, bf1_ref, wf2_ref,
             bf2_ref, out_ref):
    agg = (p_ref[0] + p_ref[1])[:, :F2] + ht_ref[:, :F2]
    h = dinv_ref[...] * agg + bc2_ref[...]
    h = jnp.maximum(h, 0.0)
    ctx = jnp.sum(h, axis=0, keepdims=True) * (1.0 / NPG)
    se = jnp.concatenate(
        [h[0:1], h[250:251], h[500:501], h[999:1000]], axis=0)
    comb = jnp.concatenate([se, jnp.broadcast_to(ctx, (4, F2))], axis=1)
    a = jnp.dot(comb.astype(jnp.bfloat16), w1t_ref[...].astype(jnp.bfloat16),
                preferred_element_type=jnp.float32)
    a = jnp.maximum(a + bf1_ref[...], 0.0)
    ab = a.astype(jnp.bfloat16).astype(jnp.float32)
    wb = wf2_ref[...].astype(jnp.bfloat16).astype(jnp.float32)
    q = jnp.sum(ab * wb, axis=1)
    g = pl.program_id(0)
    out_ref[pl.ds(g, 1), :] = q[None, :] + bf2_ref[...]


def kernel(x, edge_index, batch, Wc1, bc1, Wc2, bc2, Wf1, bf1, Wf2, bf2):
    del batch
    src = edge_index[0]
    dst = edge_index[1]
    pad = ETOT - E
    ar = jnp.arange(pad, dtype=jnp.int32)
    srcp = jnp.concatenate([src, ar % 128]).reshape(NTILES, NCHUNK, CHUNK)
    dstp = jnp.concatenate([dst, N + (ar % 128)]).reshape(NTILES, NCHUNK, CHUNK)
    zeros1 = jnp.zeros((NPAD,), jnp.float32)
    zerosa = jnp.zeros((NPAD, FA), jnp.float32)

    degp = _make_deg_kernel()(dstp, zeros1)

    ht1, dinv = pl.pallas_call(
        _k2_body,
        grid=(G,),
        in_specs=[
            pl.BlockSpec((NPG, F0), lambda g: (g, 0)),
            pl.BlockSpec((NCORES, NPG, 1), lambda g: (0, g, 0)),
            pl.BlockSpec((F0, F1), lambda g: (0, 0)),
        ],
        out_specs=[
            pl.BlockSpec((NPG, FW), lambda g: (g, 0)),
            pl.BlockSpec((NPG, 1), lambda g: (g, 0)),
        ],
        out_shape=[
            jax.ShapeDtypeStruct((N, FW), jnp.float32),
            jax.ShapeDtypeStruct((N, 1), jnp.float32),
        ],
    )(x, degp.reshape(NCORES, NPAD, 1), Wc1)

    p1 = _make_scatter_kernel()(srcp, dstp, ht1, zerosa)

    ht2 = pl.pallas_call(
        _k4_body,
        grid=(G,),
        in_specs=[
            pl.BlockSpec((NCORES, NPG, FA), lambda g: (0, g, 0)),
            pl.BlockSpec((NPG, FW), lambda g: (g, 0)),
            pl.BlockSpec((NPG, 1), lambda g: (g, 0)),
            pl.BlockSpec((1, F1), lambda g: (0, 0)),
            pl.BlockSpec((F1, F2), lambda g: (0, 0)),
        ],
        out_specs=pl.BlockSpec((NPG, FW), lambda g: (g, 0)),
        out_shape=jax.ShapeDtypeStruct((N, FW), jnp.float32),
    )(p1, ht1, dinv, bc1.reshape(1, F1), Wc2)

    p2 = _make_scatter_kernel()(srcp, dstp, ht2, zerosa)

    out = pl.pallas_call(
        _k6_body,
        grid=(G,),
        in_specs=[
            pl.BlockSpec((NCORES, NPG, FA), lambda g: (0, g, 0)),
            pl.BlockSpec((NPG, FW), lambda g: (g, 0)),
            pl.BlockSpec((NPG, 1), lambda g: (g, 0)),
            pl.BlockSpec((1, F2), lambda g: (0, 0)),
            pl.BlockSpec((F0, F2), lambda g: (0, 0)),
            pl.BlockSpec((1, F2), lambda g: (0, 0)),
            pl.BlockSpec((1, F2), lambda g: (0, 0)),
            pl.BlockSpec((1, 1), lambda g: (0, 0)),
        ],
        out_specs=pl.BlockSpec((G, 4), lambda g: (0, 0)),
        out_shape=jax.ShapeDtypeStruct((G, 4), jnp.float32),
    )(p2, ht2, dinv, bc2.reshape(1, F2), Wf1.T, bf1.reshape(1, F2),
      Wf2, bf2.reshape(1, 1))

    return out
